# baseline (speedup 1.0000x reference)
import functools

import jax
import jax.numpy as jnp
from jax import lax
from jax.experimental import pallas as pl
from jax.experimental.pallas import tpu as pltpu
from jax.experimental.pallas import tpu_sc as plsc

_N_FIELDS = 26
_VOCAB = 100000
_DIM = 16
_BATCH = 16384
_LANES = 16

_NC = 2
_NS = 16
_NW = _NC * _NS
_BW = _BATCH // _NW
_CH = 128
_NCHUNK = _BW // _CH

_GRP = 8
_ROWS = _VOCAB // _GRP
_NBLK = _N_FIELDS * _ROWS
_FPB = 2
_TCG = _N_FIELDS // _FPB


def _relayout_body(a_hbm, out_ref, b0, b1, sem):
    i = pl.program_id(0)
    bufs = (b0, b1)
    eye = jnp.eye(_DIM, dtype=jnp.bfloat16)

    @pl.when(i == 0)
    def _():
        pltpu.async_copy(a_hbm.at[pl.ds(0, 1)], b0, sem)

    for fi in range(_FPB):
        f = i * _FPB + fi
        nxt = jnp.minimum(f + 1, _N_FIELDS - 1)
        pltpu.async_copy(a_hbm.at[pl.ds(nxt, 1)], bufs[1 - fi], sem)
        pltpu.make_async_copy(a_hbm.at[pl.ds(0, 1)], bufs[fi], sem).wait()
        acc = None
        for g in range(_GRP):
            piece = bufs[fi][0, :, pl.ds(g * _ROWS, _ROWS)]
            sel = jnp.pad(eye, ((0, 0), (g * _DIM, 128 - (g + 1) * _DIM)))
            part = lax.dot_general(
                piece.astype(jnp.bfloat16), sel, (((0,), (0,)), ((), ())),
                preferred_element_type=jnp.float32)
            acc = part if acc is None else acc + part
        out_ref[pl.ds(fi * _ROWS, _ROWS), :] = acc

    @pl.when(i == _TCG - 1)
    def _():
        pltpu.make_async_copy(a_hbm.at[pl.ds(0, 1)], b0, sem).wait()


@functools.lru_cache(maxsize=None)
def _build_relayout():
    return pl.pallas_call(
        _relayout_body,
        grid=(_TCG,),
        in_specs=[pl.BlockSpec(memory_space=pl.ANY)],
        out_specs=pl.BlockSpec((_FPB * _ROWS, _GRP * _DIM), lambda i: (i, 0)),
        out_shape=jax.ShapeDtypeStruct((_NBLK, _GRP * _DIM), jnp.float32),
        scratch_shapes=[
            pltpu.VMEM((1, _DIM, _VOCAB), jnp.float32),
            pltpu.VMEM((1, _DIM, _VOCAB), jnp.float32),
            pltpu.SemaphoreType.DMA,
        ],
    )


def _emb_body(tab_hbm, xt_hbm, out_hbm, xv, rows_v, out_v, sem):
    wid = lax.axis_index("s") * _NC + lax.axis_index("c")
    base = wid * _BW
    pltpu.sync_copy(xt_hbm.at[:, pl.ds(base, _BW)], xv)

    def chunk(g, carry):
        copies = []
        for f in range(_N_FIELDS):
            copies.append(pltpu.async_copy(
                tab_hbm.at[f].at[xv.at[f, pl.ds(g * _CH, _CH)]],
                rows_v.at[f],
                sem,
            ))
        for cp in copies:
            cp.wait()

        def accum(i, c):
            acc = rows_v[0, i, :]
            for f in range(1, _N_FIELDS):
                acc = acc + rows_v[f, i, :]
            out_v[g * _CH + i, :] = acc
            return c

        lax.fori_loop(0, _CH, accum, 0)
        return carry

    lax.fori_loop(0, _NCHUNK, chunk, 0)
    pltpu.sync_copy(out_v, out_hbm.at[pl.ds(base, _BW)])


@functools.lru_cache(maxsize=None)
def _build_emb():
    return functools.partial(
        pl.kernel,
        out_type=jax.ShapeDtypeStruct((_BATCH, _DIM), jnp.float32),
        mesh=plsc.VectorSubcoreMesh(core_axis_name="c", subcore_axis_name="s"),
        compiler_params=pltpu.CompilerParams(use_tc_tiling_on_sc=False),
        scratch_types=[
            pltpu.VMEM((_N_FIELDS, _BW), jnp.int32),
            pltpu.VMEM((_N_FIELDS, _CH, _DIM), jnp.float32),
            pltpu.VMEM((_BW, _DIM), jnp.float32),
            pltpu.SemaphoreType.DMA,
        ],
    )(_emb_body)


@jax.jit
def kernel(x, tables):
    tab_rows = _build_relayout()(tables.transpose(0, 2, 1))
    tab_flat = tab_rows.reshape(_N_FIELDS, _VOCAB, _DIM)
    xm = (x % _ROWS) * _GRP + x // _ROWS
    return _build_emb()(tab_flat, xm.T)


# speedup vs baseline: 2.0297x; 2.0297x over previous
import functools

import jax
import jax.numpy as jnp
from jax import lax
from jax.experimental import pallas as pl
from jax.experimental.pallas import tpu as pltpu
from jax.experimental.pallas import tpu_sc as plsc

_N_FIELDS = 26
_VOCAB = 100000
_DIM = 16
_BATCH = 16384
_LANES = 16

_NC = 2
_NS = 16
_NW = _NC * _NS
_BW = _BATCH // _NW
_CH = 128
_NCHUNK = _BW // _CH

_GRP = 8
_ROWS = _VOCAB // _GRP
_NBLK = _N_FIELDS * _ROWS
_FPB = 2
_TCG = _N_FIELDS // _FPB


def _relayout_body(a_hbm, out_ref, a_ref, sem):
    i = pl.program_id(0)
    pltpu.async_copy(a_hbm.at[pl.ds(i * _FPB, _FPB)], a_ref, sem).wait()
    eye = jnp.eye(_GRP * _DIM, dtype=jnp.bfloat16)
    for fi in range(_FPB):
        stack = jnp.concatenate(
            [a_ref[fi, :, pl.ds(g * _ROWS, _ROWS)] for g in range(_GRP)],
            axis=0).astype(jnp.bfloat16)
        out_ref[pl.ds(fi * _ROWS, _ROWS), :] = lax.dot_general(
            stack, eye, (((0,), (0,)), ((), ())),
            preferred_element_type=jnp.float32)


@functools.lru_cache(maxsize=None)
def _build_relayout():
    return pl.pallas_call(
        _relayout_body,
        grid=(_TCG,),
        in_specs=[pl.BlockSpec(memory_space=pl.ANY)],
        out_specs=pl.BlockSpec((_FPB * _ROWS, _GRP * _DIM), lambda i: (i, 0)),
        out_shape=jax.ShapeDtypeStruct((_NBLK, _GRP * _DIM), jnp.float32),
        scratch_shapes=[
            pltpu.VMEM((_FPB, _DIM, _VOCAB), jnp.float32),
            pltpu.SemaphoreType.DMA,
        ],
    )


def _emb_body(tab_hbm, xt_hbm, out_hbm, xv, rows_v, out_v, sem):
    wid = lax.axis_index("s") * _NC + lax.axis_index("c")
    base = wid * _BW
    pltpu.sync_copy(xt_hbm.at[:, pl.ds(base, _BW)], xv)

    def chunk(g, carry):
        copies = []
        for f in range(_N_FIELDS):
            copies.append(pltpu.async_copy(
                tab_hbm.at[f].at[xv.at[f, pl.ds(g * _CH, _CH)]],
                rows_v.at[f],
                sem,
            ))
        for cp in copies:
            cp.wait()

        def accum(i, c):
            acc = rows_v[0, i, :]
            for f in range(1, _N_FIELDS):
                acc = acc + rows_v[f, i, :]
            out_v[g * _CH + i, :] = acc
            return c

        lax.fori_loop(0, _CH, accum, 0)
        return carry

    lax.fori_loop(0, _NCHUNK, chunk, 0)
    pltpu.sync_copy(out_v, out_hbm.at[pl.ds(base, _BW)])


@functools.lru_cache(maxsize=None)
def _build_emb():
    return functools.partial(
        pl.kernel,
        out_type=jax.ShapeDtypeStruct((_BATCH, _DIM), jnp.float32),
        mesh=plsc.VectorSubcoreMesh(core_axis_name="c", subcore_axis_name="s"),
        compiler_params=pltpu.CompilerParams(use_tc_tiling_on_sc=False),
        scratch_types=[
            pltpu.VMEM((_N_FIELDS, _BW), jnp.int32),
            pltpu.VMEM((_N_FIELDS, _CH, _DIM), jnp.float32),
            pltpu.VMEM((_BW, _DIM), jnp.float32),
            pltpu.SemaphoreType.DMA,
        ],
    )(_emb_body)


@jax.jit
def kernel(x, tables):
    tab_rows = _build_relayout()(tables.transpose(0, 2, 1))
    tab_flat = tab_rows.reshape(_N_FIELDS, _VOCAB, _DIM)
    xm = (x % _ROWS) * _GRP + x // _ROWS
    return _build_emb()(tab_flat, xm.T)


# speedup vs baseline: 2.8675x; 1.4127x over previous
import functools

import jax
import jax.numpy as jnp
from jax import lax
from jax.experimental import pallas as pl
from jax.experimental.pallas import tpu as pltpu
from jax.experimental.pallas import tpu_sc as plsc

_N_FIELDS = 26
_VOCAB = 100000
_DIM = 16
_BATCH = 16384
_LANES = 16

_NC = 2
_NS = 16
_NW = _NC * _NS
_BW = _BATCH // _NW
_CH = 128
_NCHUNK = _BW // _CH

_GRP = 8
_ROWS = _VOCAB // _GRP
_NBLK = _N_FIELDS * _ROWS
_FPB = 2
_TCG = _N_FIELDS // _FPB


def _relayout_body(a_ref, out_ref):
    eye = jnp.eye(_GRP * _DIM, dtype=jnp.bfloat16)
    for fi in range(_FPB):
        stack = jnp.concatenate(
            [a_ref[fi, :, pl.ds(g * _ROWS, _ROWS)] for g in range(_GRP)],
            axis=0).astype(jnp.bfloat16)
        out_ref[pl.ds(fi * _ROWS, _ROWS), :] = lax.dot_general(
            stack, eye, (((0,), (0,)), ((), ())),
            preferred_element_type=jnp.float32)


@functools.lru_cache(maxsize=None)
def _build_relayout():
    return pl.pallas_call(
        _relayout_body,
        grid=(_TCG,),
        in_specs=[pl.BlockSpec((_FPB, _DIM, _VOCAB), lambda i: (i, 0, 0))],
        out_specs=pl.BlockSpec((_FPB * _ROWS, _GRP * _DIM), lambda i: (i, 0)),
        out_shape=jax.ShapeDtypeStruct((_NBLK, _GRP * _DIM), jnp.float32),
    )


def _emb_body(tab_hbm, xt_hbm, out_hbm, xv, rows_v, out_v, sem):
    wid = lax.axis_index("s") * _NC + lax.axis_index("c")
    base = wid * _BW
    pltpu.sync_copy(xt_hbm.at[:, pl.ds(base, _BW)], xv)

    def chunk(g, carry):
        copies = []
        for f in range(_N_FIELDS):
            copies.append(pltpu.async_copy(
                tab_hbm.at[f].at[xv.at[f, pl.ds(g * _CH, _CH)]],
                rows_v.at[f],
                sem,
            ))
        for cp in copies:
            cp.wait()

        def accum(i, c):
            acc = rows_v[0, i, :]
            for f in range(1, _N_FIELDS):
                acc = acc + rows_v[f, i, :]
            out_v[g * _CH + i, :] = acc
            return c

        lax.fori_loop(0, _CH, accum, 0)
        return carry

    lax.fori_loop(0, _NCHUNK, chunk, 0)
    pltpu.sync_copy(out_v, out_hbm.at[pl.ds(base, _BW)])


@functools.lru_cache(maxsize=None)
def _build_emb():
    return functools.partial(
        pl.kernel,
        out_type=jax.ShapeDtypeStruct((_BATCH, _DIM), jnp.float32),
        mesh=plsc.VectorSubcoreMesh(core_axis_name="c", subcore_axis_name="s"),
        compiler_params=pltpu.CompilerParams(use_tc_tiling_on_sc=False),
        scratch_types=[
            pltpu.VMEM((_N_FIELDS, _BW), jnp.int32),
            pltpu.VMEM((_N_FIELDS, _CH, _DIM), jnp.float32),
            pltpu.VMEM((_BW, _DIM), jnp.float32),
            pltpu.SemaphoreType.DMA,
        ],
    )(_emb_body)


@jax.jit
def kernel(x, tables):
    tab_rows = _build_relayout()(tables.transpose(0, 2, 1))
    tab_flat = tab_rows.reshape(_N_FIELDS, _VOCAB, _DIM)
    xm = (x % _ROWS) * _GRP + x // _ROWS
    return _build_emb()(tab_flat, xm.T)
